# Initial kernel scaffold; baseline (speedup 1.0000x reference)
import jax
import jax.numpy as jnp
from jax.experimental import pallas as pl


def kernel(x, edge_index, edge_attr, labels, Wl1, bl1, Wr1, br1, We1, att1, b1, Wl2, bl2, Wr2, br2, We2, att2, b2, Wfc, bfc):
    raise NotImplementedError("write your pallas kernel here")


# speedup vs baseline: 783.5352x; 783.5352x over previous
import functools

import jax
import jax.numpy as jnp
from jax import lax
from jax.experimental import pallas as pl
from jax.experimental.pallas import tpu as pltpu
from jax.experimental.pallas import tpu_sc as plsc

_N = 50000
_E = 800000
_EMB = 16
_H = 2
_HID = 64
_HE = _H * _EMB
_NW = 32
_EPW = 25008
_CH = _EPW // 16
_EPAD = _NW * _EPW

_NEG = -1e30


def _sc_kernel_factory():
    mesh = plsc.VectorSubcoreMesh(core_axis_name="c", subcore_axis_name="s")

    @functools.partial(
        pl.kernel,
        mesh=mesh,
        out_type=jax.ShapeDtypeStruct((_NW, 6, 16), jnp.float32),
        compiler_params=pltpu.CompilerParams(needs_layout_passes=False),
        scratch_types=[
            pltpu.VMEM((_EPW,), jnp.int32),
            pltpu.VMEM((_EPW + 16,), jnp.int32),
            pltpu.VMEM((18, 16), jnp.float32),
            pltpu.VMEM((16,), jnp.int32),
            pltpu.VMEM((16,), jnp.int32),
            pltpu.VMEM((16,), jnp.float32),
            pltpu.VMEM((16,), jnp.float32),
            pltpu.VMEM((16,), jnp.float32),
            pltpu.VMEM((6, 16), jnp.float32),
            pltpu.SemaphoreType.DMA,
        ],
    )
    def sck(dst_hbm, src_hbm, ea_hbm, x0_hbm, x1_hbm, w_hbm, out_hbm,
            dst_v, buf_v, w_v, eid_v, srcg_v, eag_v, x0g_v, x1g_v,
            part_v, sem):
        cid = lax.axis_index("c")
        sid = lax.axis_index("s")
        wid = sid * 2 + cid
        base = wid * _EPW

        pltpu.sync_copy(dst_hbm.at[pl.ds(base, _EPW)], dst_v)
        pltpu.sync_copy(w_hbm, w_v)

        iota = lax.iota(jnp.int32, 16)
        zero16i = jnp.zeros((16,), jnp.int32)

        def scan_body(i, cnt):
            v = dst_v[pl.ds(i * 16, 16)]
            msk = v == 0
            cs = plsc.cumsum(msk.astype(jnp.int32))
            eids = (base + i * 16) + iota
            plsc.store_scatter(buf_v, [cnt + cs - 1], eids, mask=msk)
            return cnt + jnp.max(cs)

        cnt = lax.fori_loop(0, _CH, scan_body, jnp.int32(0))
        plsc.store_scatter(buf_v, [cnt + iota], zero16i, mask=iota >= 0)

        x00 = w_v[0]
        x01 = w_v[1]
        xr00 = x00 * w_v[6] + x01 * w_v[8] + w_v[12]
        xr01 = x00 * w_v[7] + x01 * w_v[9] + w_v[13]
        att0 = w_v[16]
        att1 = w_v[17]

        def chunk_body(c, state):
            cb = c * 16
            eid_v[...] = buf_v[pl.ds(cb, 16)]
            pltpu.async_copy(src_hbm.at[eid_v], srcg_v, sem).wait()
            pltpu.async_copy(ea_hbm.at[eid_v], eag_v, sem).wait()
            pltpu.async_copy(x0_hbm.at[srcg_v], x0g_v, sem).wait()
            pltpu.async_copy(x1_hbm.at[srcg_v], x1g_v, sem).wait()
            rem = jnp.minimum(cnt - cb, 16)

            def edge_body(j, st):
                m0, m1, d0, d1, n0, n1 = st
                jj = jnp.full((16,), j, jnp.int32)
                xs0 = plsc.load_gather(x0g_v, [jj])
                xs1 = plsc.load_gather(x1g_v, [jj])
                eas = plsc.load_gather(eag_v, [jj])
                xl0 = xs0 * w_v[2] + xs1 * w_v[4] + w_v[10]
                xl1 = xs0 * w_v[3] + xs1 * w_v[5] + w_v[11]
                z0 = xl0 + xr00 + eas * w_v[14]
                z1 = xl1 + xr01 + eas * w_v[15]
                lr0 = jnp.maximum(z0, 0.2 * z0)
                lr1 = jnp.maximum(z1, 0.2 * z1)
                a0 = jnp.full((16,), jnp.sum(lr0 * att0), jnp.float32)
                a1 = jnp.full((16,), jnp.sum(lr1 * att1), jnp.float32)
                nm0 = jnp.maximum(m0, a0)
                nm1 = jnp.maximum(m1, a1)
                co0 = jnp.exp(m0 - nm0)
                cn0 = jnp.exp(a0 - nm0)
                co1 = jnp.exp(m1 - nm1)
                cn1 = jnp.exp(a1 - nm1)
                return (nm0, nm1,
                        d0 * co0 + cn0, d1 * co1 + cn1,
                        n0 * co0 + cn0 * xl0, n1 * co1 + cn1 * xl1)

            return lax.fori_loop(0, rem, edge_body, state)

        zf = jnp.zeros((16,), jnp.float32)
        neg = jnp.full((16,), _NEG, jnp.float32)
        nch = (cnt + 15) // 16
        m0, m1, d0, d1, n0, n1 = lax.fori_loop(
            0, nch, chunk_body, (neg, neg, zf, zf, zf, zf))

        part_v[0] = m0
        part_v[1] = m1
        part_v[2] = d0
        part_v[3] = d1
        part_v[4] = n0
        part_v[5] = n1
        pltpu.sync_copy(part_v, out_hbm.at[wid])

    return sck


def _tc_combine(p_ref, b2_ref, wfc_ref, bfc_ref, o_ref):
    p = p_ref[...]
    m0 = p[:, 0:16]
    m1 = p[:, 16:32]
    d0 = p[:, 32:48]
    d1 = p[:, 48:64]
    n0 = p[:, 64:80]
    n1 = p[:, 80:96]
    mx0 = jnp.max(m0, axis=0, keepdims=True)
    mx1 = jnp.max(m1, axis=0, keepdims=True)
    w0 = jnp.exp(m0 - mx0)
    w1 = jnp.exp(m1 - mx1)
    den0 = jnp.sum(d0 * w0, axis=0, keepdims=True)
    den1 = jnp.sum(d1 * w1, axis=0, keepdims=True)
    num0 = jnp.sum(n0 * w0, axis=0, keepdims=True)
    num1 = jnp.sum(n1 * w1, axis=0, keepdims=True)
    te = jnp.concatenate(
        [num0 / (den0 + 1e-16), num1 / (den1 + 1e-16)], axis=1)
    h0 = jnp.maximum(te + b2_ref[...], 0.0)
    o_ref[...] = lax.dot_general(
        h0, wfc_ref[...], (((1,), (0,)), ((), ())),
        precision=lax.Precision.HIGHEST,
        preferred_element_type=jnp.float32) + bfc_ref[...]


def kernel(x, edge_index, edge_attr, labels,
           Wl1, bl1, Wr1, br1, We1, att1, b1,
           Wl2, bl2, Wr2, br2, We2, att2, b2,
           Wfc, bfc):
    src = edge_index[0]
    dstp = jnp.concatenate(
        [edge_index[1], jnp.ones((_EPAD - _E,), jnp.int32)])
    ea = edge_attr[:, 0]
    x0 = x[:, 0]
    x1 = x[:, 1]

    wl = Wl2.reshape(2, _H, _EMB)
    wr = Wr2.reshape(2, _H, _EMB)
    wpack = jnp.stack([
        jnp.full((16,), x[0, 0], jnp.float32),
        jnp.full((16,), x[0, 1], jnp.float32),
        wl[0, 0], wl[0, 1], wl[1, 0], wl[1, 1],
        wr[0, 0], wr[0, 1], wr[1, 0], wr[1, 1],
        bl2.reshape(_H, _EMB)[0], bl2.reshape(_H, _EMB)[1],
        br2.reshape(_H, _EMB)[0], br2.reshape(_H, _EMB)[1],
        We2.reshape(_H, _EMB)[0], We2.reshape(_H, _EMB)[1],
        att2[0], att2[1],
    ])

    parts = _sc_kernel_factory()(dstp, src, ea, x0, x1, wpack)
    out = pl.pallas_call(
        _tc_combine,
        out_shape=jax.ShapeDtypeStruct((1, _HID), jnp.float32),
    )(parts.reshape(_NW, 6 * 16), b2.reshape(1, _HE),
      Wfc, bfc.reshape(1, _HID))
    return out


# speedup vs baseline: 863.0034x; 1.1014x over previous
import functools

import jax
import jax.numpy as jnp
from jax import lax
from jax.experimental import pallas as pl
from jax.experimental.pallas import tpu as pltpu
from jax.experimental.pallas import tpu_sc as plsc

_N = 50000
_E = 800000
_EMB = 16
_H = 2
_HID = 64
_HE = _H * _EMB
_NW = 16
_EPW = _E // _NW
_CH = _EPW // 16
_GRP = 8
_NG = _CH // _GRP

_NEG = -1e30

_R_X0, _R_X1 = 0, 1
_R_WL = 2
_R_WR = 6
_R_BL = 10
_R_BR = 12
_R_WE = 14
_R_ATT = 16
_R_B2 = 18
_R_BFC = 20
_R_WFC = 24
_WROWS = 24 + _HE * 4


def _sc_kernel_factory():
    mesh = plsc.VectorSubcoreMesh(
        core_axis_name="c", subcore_axis_name="s", num_cores=1)

    @functools.partial(
        pl.kernel,
        mesh=mesh,
        out_type=jax.ShapeDtypeStruct((4, 16), jnp.float32),
        compiler_params=pltpu.CompilerParams(
            needs_layout_passes=False, use_tc_tiling_on_sc=False),
        scratch_types=[
            pltpu.VMEM((_EPW,), jnp.int32),
            pltpu.VMEM((_EPW + 16,), jnp.int32),
            pltpu.VMEM((_WROWS, 16), jnp.float32),
            pltpu.VMEM((16,), jnp.int32),
            pltpu.VMEM((16,), jnp.int32),
            pltpu.VMEM((16,), jnp.float32),
            pltpu.VMEM((16,), jnp.float32),
            pltpu.VMEM((16,), jnp.float32),
            pltpu.VMEM((6, 16), jnp.float32),
            pltpu.VMEM_SHARED((_NW, 6, 16), jnp.float32),
            pltpu.VMEM((_NW, 6, 16), jnp.float32),
            pltpu.VMEM((4, 16), jnp.float32),
            pltpu.SemaphoreType.DMA,
            pltpu.SemaphoreType.DMA,
        ],
    )
    def sck(dst_hbm, src_hbm, ea_hbm, x0_hbm, x1_hbm, w_hbm, out_hbm,
            dst_v, buf_v, w_v, eid_v, srcg_v, eag_v, x0g_v, x1g_v,
            part_v, shr_v, comb_v, out_v, sem, sem2):
        wid = lax.axis_index("s")
        base = wid * _EPW

        cp_dst = pltpu.async_copy(dst_hbm.at[pl.ds(base, _EPW)], dst_v, sem)
        cp_w = pltpu.async_copy(w_hbm, w_v, sem2)
        cp_dst.wait()
        cp_w.wait()

        iota = lax.iota(jnp.int32, 16)

        def scan_chunk(c, cnt):
            v = dst_v[pl.ds(c * 16, 16)]
            msk = v == 0
            cs = plsc.cumsum(msk.astype(jnp.int32))
            eids = (base + c * 16) + iota
            plsc.store_scatter(buf_v, [cnt + cs - 1], eids, mask=msk)
            return cnt + jnp.max(cs)

        def group_body(g, cnt):
            c0 = g * _GRP
            mn = dst_v[pl.ds(c0 * 16, 16)]
            for k in range(1, _GRP):
                mn = jnp.minimum(mn, dst_v[pl.ds((c0 + k) * 16, 16)])
            return lax.cond(
                jnp.min(mn) == 0,
                lambda c: lax.fori_loop(c0, c0 + _GRP, scan_chunk, c),
                lambda c: c,
                cnt)

        cnt = lax.fori_loop(0, _NG, group_body, jnp.int32(0))
        cnt = lax.fori_loop(_NG * _GRP, _CH, scan_chunk, cnt)
        plsc.store_scatter(buf_v, [cnt + iota],
                           jnp.zeros((16,), jnp.int32), mask=iota >= 0)

        xr00 = w_v[_R_X0] * w_v[_R_WR] + w_v[_R_X1] * w_v[_R_WR + 2] + w_v[_R_BR]
        xr01 = w_v[_R_X0] * w_v[_R_WR + 1] + w_v[_R_X1] * w_v[_R_WR + 3] + w_v[_R_BR + 1]
        att0 = w_v[_R_ATT]
        att1 = w_v[_R_ATT + 1]

        def chunk_body(c, state):
            cb = c * 16
            eid_v[...] = buf_v[pl.ds(cb, 16)]
            cpa = pltpu.async_copy(src_hbm.at[eid_v], srcg_v, sem)
            cpb = pltpu.async_copy(ea_hbm.at[eid_v], eag_v, sem2)
            cpa.wait()
            cpb.wait()
            cpc = pltpu.async_copy(x0_hbm.at[srcg_v], x0g_v, sem)
            cpd = pltpu.async_copy(x1_hbm.at[srcg_v], x1g_v, sem2)
            cpc.wait()
            cpd.wait()
            rem = jnp.minimum(cnt - cb, 16)

            def edge_body(j, st):
                m0, m1, d0, d1, n0, n1 = st
                jj = jnp.full((16,), j, jnp.int32)
                xs0 = plsc.load_gather(x0g_v, [jj])
                xs1 = plsc.load_gather(x1g_v, [jj])
                eas = plsc.load_gather(eag_v, [jj])
                xl0 = xs0 * w_v[_R_WL] + xs1 * w_v[_R_WL + 2] + w_v[_R_BL]
                xl1 = xs0 * w_v[_R_WL + 1] + xs1 * w_v[_R_WL + 3] + w_v[_R_BL + 1]
                z0 = xl0 + xr00 + eas * w_v[_R_WE]
                z1 = xl1 + xr01 + eas * w_v[_R_WE + 1]
                lr0 = jnp.maximum(z0, 0.2 * z0)
                lr1 = jnp.maximum(z1, 0.2 * z1)
                a0 = jnp.full((16,), jnp.sum(lr0 * att0), jnp.float32)
                a1 = jnp.full((16,), jnp.sum(lr1 * att1), jnp.float32)
                nm0 = jnp.maximum(m0, a0)
                nm1 = jnp.maximum(m1, a1)
                co0 = jnp.exp(m0 - nm0)
                cn0 = jnp.exp(a0 - nm0)
                co1 = jnp.exp(m1 - nm1)
                cn1 = jnp.exp(a1 - nm1)
                return (nm0, nm1,
                        d0 * co0 + cn0, d1 * co1 + cn1,
                        n0 * co0 + cn0 * xl0, n1 * co1 + cn1 * xl1)

            return lax.fori_loop(0, rem, edge_body, state)

        zf = jnp.zeros((16,), jnp.float32)
        neg = jnp.full((16,), _NEG, jnp.float32)
        nch = (cnt + 15) // 16
        m0, m1, d0, d1, n0, n1 = lax.fori_loop(
            0, nch, chunk_body, (neg, neg, zf, zf, zf, zf))

        part_v[0] = m0
        part_v[1] = m1
        part_v[2] = d0
        part_v[3] = d1
        part_v[4] = n0
        part_v[5] = n1
        pltpu.sync_copy(part_v, shr_v.at[wid])
        plsc.subcore_barrier()

        @pl.when(wid == 0)
        def _():
            pltpu.sync_copy(shr_v, comb_v)
            M0 = comb_v[0, 0]
            M1 = comb_v[0, 1]
            D0 = comb_v[0, 2]
            D1 = comb_v[0, 3]
            N0 = comb_v[0, 4]
            N1 = comb_v[0, 5]
            for w in range(1, _NW):
                mw0 = comb_v[w, 0]
                mw1 = comb_v[w, 1]
                nm0 = jnp.maximum(M0, mw0)
                nm1 = jnp.maximum(M1, mw1)
                co0 = jnp.exp(M0 - nm0)
                cw0 = jnp.exp(mw0 - nm0)
                co1 = jnp.exp(M1 - nm1)
                cw1 = jnp.exp(mw1 - nm1)
                D0 = D0 * co0 + comb_v[w, 2] * cw0
                D1 = D1 * co1 + comb_v[w, 3] * cw1
                N0 = N0 * co0 + comb_v[w, 4] * cw0
                N1 = N1 * co1 + comb_v[w, 5] * cw1
                M0 = nm0
                M1 = nm1
            h00 = jnp.maximum(N0 / (D0 + 1e-16) + w_v[_R_B2], 0.0)
            h01 = jnp.maximum(N1 / (D1 + 1e-16) + w_v[_R_B2 + 1], 0.0)
            acc = [w_v[_R_BFC + j] for j in range(4)]
            for i in range(_HE):
                hrow = h00 if i < 16 else h01
                sel = jnp.where(iota == (i % 16), hrow,
                                jnp.zeros((16,), jnp.float32))
                s = jnp.full((16,), jnp.sum(sel), jnp.float32)
                for j in range(4):
                    acc[j] = acc[j] + s * w_v[_R_WFC + i * 4 + j]
            for j in range(4):
                out_v[j] = acc[j]
            pltpu.sync_copy(out_v, out_hbm)

    return sck


def kernel(x, edge_index, edge_attr, labels,
           Wl1, bl1, Wr1, br1, We1, att1, b1,
           Wl2, bl2, Wr2, br2, We2, att2, b2,
           Wfc, bfc):
    src = edge_index[0]
    dst = edge_index[1]
    ea = edge_attr.reshape(_E)
    x0 = x[:, 0]
    x1 = x[:, 1]

    wl = Wl2.reshape(2, _H, _EMB)
    wr = Wr2.reshape(2, _H, _EMB)
    head = jnp.stack([
        jnp.full((16,), x[0, 0], jnp.float32),
        jnp.full((16,), x[0, 1], jnp.float32),
        wl[0, 0], wl[0, 1], wl[1, 0], wl[1, 1],
        wr[0, 0], wr[0, 1], wr[1, 0], wr[1, 1],
        bl2.reshape(_H, _EMB)[0], bl2.reshape(_H, _EMB)[1],
        br2.reshape(_H, _EMB)[0], br2.reshape(_H, _EMB)[1],
        We2.reshape(_H, _EMB)[0], We2.reshape(_H, _EMB)[1],
        att2[0], att2[1],
        b2.reshape(_H, _EMB)[0], b2.reshape(_H, _EMB)[1],
    ])
    wpack = jnp.concatenate(
        [head, bfc.reshape(4, 16), Wfc.reshape(_HE * 4, 16)])

    out = _sc_kernel_factory()(dst, src, ea, x0, x1, wpack)
    return out.reshape(1, _HID)
